# baseline (speedup 1.0000x reference)
import functools

import jax
import jax.numpy as jnp
from jax import lax
from jax.experimental import pallas as pl
from jax.experimental.pallas import tpu as pltpu
from jax.experimental.pallas import tpu_sc as plsc

NC, NS = 2, 16
NW = NC * NS
BATCH, N_FIELDS = 16384, 26
D = 64
B = BATCH * N_FIELDS
BPW = B // NW
C = 128
NCHUNK = BPW // C
GC = 4
RG = GC * C
NGROUP = NCHUNK // GC
NBUF = 2

_mesh = plsc.VectorSubcoreMesh(
    core_axis_name="c", subcore_axis_name="s", num_cores=NC, num_subcores=NS)


@functools.partial(
    pl.kernel,
    out_type=jax.ShapeDtypeStruct((B, D), jnp.float32),
    mesh=_mesh,
    scratch_types=[
        pltpu.VMEM((NGROUP, RG), jnp.int32),
        pltpu.VMEM((RG, D), jnp.float32),
        pltpu.VMEM((RG, D), jnp.float32),
        pltpu.SemaphoreType.DMA,
        pltpu.SemaphoreType.DMA,
        pltpu.SemaphoreType.DMA,
        pltpu.SemaphoreType.DMA,
    ],
    compiler_params=pltpu.CompilerParams(use_tc_tiling_on_sc=False),
)
def _gather(ids_hbm, table_hbm, out_hbm, idx_v, rows0, rows1, gs0, gs1,
            ws0, ws1):
    wid = lax.axis_index("s") * NC + lax.axis_index("c")
    base = wid * BPW
    pltpu.sync_copy(ids_hbm.at[wid], idx_v)

    rows = (rows0, rows1)
    gsem = (gs0, gs1)
    wsem = (ws0, ws1)

    def fire_g(g, s):
        pltpu.async_copy(table_hbm.at[idx_v.at[g]], rows[s], gsem[s])

    def drain_g(s):
        pltpu.make_async_copy(
            table_hbm.at[idx_v.at[0]], rows[s], gsem[s]).wait()

    def write(g, s):
        pltpu.async_copy(
            rows[s], out_hbm.at[pl.ds(base + g * RG, RG)], wsem[s]).wait()

    for s in range(NBUF):
        fire_g(s, s)

    @pl.loop(0, NGROUP - NBUF, step=NBUF)
    def _grp(go):
        for s in range(NBUF):
            g = go + s
            drain_g(s)
            write(g, s)
            fire_g(g + NBUF, s)

    for s in range(NBUF):
        drain_g(s)
        write(NGROUP - NBUF + s, s)


def kernel(ids, table):
    ids_w = jnp.asarray(ids, jnp.int32).reshape(NW, NGROUP, RG)
    out = _gather(ids_w, table)
    return out.reshape(BATCH, N_FIELDS, D)


# speedup vs baseline: 1.0045x; 1.0045x over previous
import functools

import jax
import jax.numpy as jnp
from jax import lax
from jax.experimental import pallas as pl
from jax.experimental.pallas import tpu as pltpu
from jax.experimental.pallas import tpu_sc as plsc

NC, NS = 2, 16
NW = NC * NS
BATCH, N_FIELDS = 16384, 26
D = 64
B = BATCH * N_FIELDS
BPW = B // NW
RG = 512
NGROUP = BPW // RG
NBUF = 3
LEAD = NBUF - 1

_mesh = plsc.VectorSubcoreMesh(
    core_axis_name="c", subcore_axis_name="s", num_cores=NC, num_subcores=NS)


@functools.partial(
    pl.kernel,
    out_type=jax.ShapeDtypeStruct((B, D), jnp.float32),
    mesh=_mesh,
    scratch_types=[
        pltpu.VMEM((NGROUP, RG), jnp.int32),
        pltpu.VMEM((RG, D), jnp.float32),
        pltpu.VMEM((RG, D), jnp.float32),
        pltpu.VMEM((RG, D), jnp.float32),
        pltpu.SemaphoreType.DMA,
        pltpu.SemaphoreType.DMA,
        pltpu.SemaphoreType.DMA,
        pltpu.SemaphoreType.DMA,
        pltpu.SemaphoreType.DMA,
        pltpu.SemaphoreType.DMA,
    ],
    compiler_params=pltpu.CompilerParams(use_tc_tiling_on_sc=False),
)
def _gather(ids_hbm, table_hbm, out_hbm, idx_v, rows0, rows1, rows2,
            gs0, gs1, gs2, ws0, ws1, ws2):
    wid = lax.axis_index("s") * NC + lax.axis_index("c")
    base = wid * BPW
    pltpu.sync_copy(ids_hbm.at[wid], idx_v)

    rows = (rows0, rows1, rows2)
    gsem = (gs0, gs1, gs2)
    wsem = (ws0, ws1, ws2)

    def fire_g(g, s):
        pltpu.async_copy(table_hbm.at[idx_v.at[g]], rows[s], gsem[s])

    def drain_g(s):
        pltpu.make_async_copy(
            table_hbm.at[idx_v.at[0]], rows[s], gsem[s]).wait()

    def fire_w(g, s):
        pltpu.async_copy(rows[s], out_hbm.at[pl.ds(base + g * RG, RG)],
                         wsem[s])

    def drain_w(g, s):
        pltpu.make_async_copy(
            rows[s], out_hbm.at[pl.ds(base + g * RG, RG)], wsem[s]).wait()

    for g in range(LEAD):
        fire_g(g, g % NBUF)
    for g in range(NGROUP - LEAD):
        gf = g + LEAD
        sf = gf % NBUF
        if gf >= NBUF:
            drain_w(gf - NBUF, sf)
        fire_g(gf, sf)
        s = g % NBUF
        drain_g(s)
        fire_w(g, s)
    for g in range(NGROUP - LEAD, NGROUP):
        s = g % NBUF
        drain_g(s)
        fire_w(g, s)
    for g in range(NGROUP - NBUF, NGROUP):
        drain_w(g, g % NBUF)


def kernel(ids, table):
    ids_w = jnp.asarray(ids, jnp.int32).reshape(NW, NGROUP, RG)
    out = _gather(ids_w, table)
    return out.reshape(BATCH, N_FIELDS, D)
